# baseline (speedup 1.0000x reference)
import functools

import jax
import jax.numpy as jnp
from jax import lax
from jax.experimental import pallas as pl
from jax.experimental.pallas import tpu as pltpu
from jax.experimental.pallas import tpu_sc as plsc

_MAXD = 128
_H = 12
_S = 2048
_T = 2 * _MAXD + 1
_TTF = _H * _T
_NC, _NS = 2, 16
_NW = _NC * _NS
_NI = _S // 8
_PAIRS = _H * _NI
_PPW = _PAIRS // _NW


@functools.partial(
    pl.kernel,
    out_type=jax.ShapeDtypeStruct((_H, _NI, 16, 8, 128), jnp.float32),
    mesh=plsc.VectorSubcoreMesh(
        core_axis_name="c", subcore_axis_name="s",
        num_cores=_NC, num_subcores=_NS,
    ),
    scratch_types=[
        pltpu.VMEM((3104,), jnp.float32),
        pltpu.VMEM((2 * 4096,), jnp.float32),
        pltpu.VMEM((2, 2, 16, 8, 128), jnp.float32),
        pltpu.SemaphoreType.DMA,
    ],
)
def _bias_kernel(tbl_hbm, out_hbm, tbl_v, st, stg, sem):
    pltpu.async_copy(tbl_hbm, tbl_v.at[pl.ds(0, _TTF)], sem).wait()

    wid = lax.axis_index("s") * _NC + lax.axis_index("c")
    n0 = wid * _PPW
    h0 = (3 * wid) // 8

    for hh in range(2):
        h = jnp.minimum(h0 + hh, _H - 1)
        tb = h * _T
        base = hh * 4096
        pre = lax.broadcast_in_dim(tbl_v[pl.ds(tb, 16)][0], (16,), ())
        suf = lax.broadcast_in_dim(tbl_v[pl.ds(tb + _T - 16, 16)][15], (16,), ())

        def prefill(j, carry, base=base, pre=pre):
            st[pl.ds(base + 16 * j, 16)] = pre
            return carry

        lax.fori_loop(0, 120, prefill, 0)

        def suffill(j, carry, base=base, suf=suf):
            st[pl.ds(base + 2176 + 16 * j, 16)] = suf
            return carry

        lax.fori_loop(0, 120, suffill, 0)

        def body(j, carry, base=base, tb=tb):
            st[pl.ds(base + 1919 + 16 * j, 16)] = tbl_v[pl.ds(tb + 16 * j, 16)]
            return carry

        lax.fori_loop(0, 17, body, 0)

        def resuf(j, carry, base=base, suf=suf):
            st[pl.ds(base + 2176 + 16 * j, 16)] = suf
            return carry

        lax.fori_loop(0, 1, resuf, 0)

    def out_body(q2, carry):
        @pl.when(q2 >= 2)
        def _():
            pltpu.make_async_copy(out_hbm.at[0, pl.ds(0, 2)], stg.at[0], sem).wait()

        n = n0 + 2 * q2
        h = n // _NI
        I = n - _NI * h
        hh = h - h0
        W0 = hh * 4096 + 2047 - 8 * I
        b = lax.rem(q2, 2)

        @plsc.parallel_loop(0, 32, step=1, unroll=2)
        def sj(t):
            ii = t // 16
            J = t - 16 * ii
            w0 = W0 - 8 * ii + 128 * J
            for r in range(8):
                for u in range(8):
                    stg[b, ii, J, r, pl.ds(16 * u, 16)] = st[pl.ds(w0 - r + 16 * u, 16)]

        pltpu.async_copy(stg.at[b], out_hbm.at[h, pl.ds(I, 2)], sem)
        return carry

    lax.fori_loop(0, _PPW // 2, out_body, 0)
    pltpu.make_async_copy(out_hbm.at[0, pl.ds(0, 2)], stg.at[0], sem).wait()
    pltpu.make_async_copy(out_hbm.at[0, pl.ds(0, 2)], stg.at[0], sem).wait()


def kernel(seq_len, table):
    del seq_len
    out = _bias_kernel(table.T.reshape(_TTF))
    return out.transpose(1, 3, 2, 4, 0).reshape(_S, _S, _H)


# speedup vs baseline: 1.0218x; 1.0218x over previous
import functools

import jax
import jax.numpy as jnp
from jax import lax
from jax.experimental import pallas as pl
from jax.experimental.pallas import tpu as pltpu
from jax.experimental.pallas import tpu_sc as plsc

_MAXD = 128
_H = 12
_S = 2048
_T = 2 * _MAXD + 1
_TTF = _H * _T
_NC, _NS = 2, 16
_NW = _NC * _NS
_NI = _S // 8
_PAIRS = _H * _NI
_PPW = _PAIRS // _NW


@functools.partial(
    pl.kernel,
    out_type=jax.ShapeDtypeStruct((_H, _NI, 16, 8, 128), jnp.float32),
    mesh=plsc.VectorSubcoreMesh(
        core_axis_name="c", subcore_axis_name="s",
        num_cores=_NC, num_subcores=_NS,
    ),
    scratch_types=[
        pltpu.VMEM((3104,), jnp.float32),
        pltpu.VMEM((2 * 4096,), jnp.float32),
        pltpu.VMEM((2, 2, 16, 8, 128), jnp.float32),
        pltpu.SemaphoreType.DMA,
    ],
)
def _bias_kernel(tbl_hbm, out_hbm, tbl_v, st, stg, sem):
    pltpu.async_copy(tbl_hbm, tbl_v.at[pl.ds(0, _TTF)], sem).wait()

    wid = lax.axis_index("s") * _NC + lax.axis_index("c")
    n0 = wid * _PPW
    h0 = (3 * wid) // 8

    for hh in range(2):
        h = jnp.minimum(h0 + hh, _H - 1)
        tb = h * _T
        base = hh * 4096
        pre = lax.broadcast_in_dim(tbl_v[pl.ds(tb, 16)][0], (16,), ())
        suf = lax.broadcast_in_dim(tbl_v[pl.ds(tb + _T - 16, 16)][15], (16,), ())

        @plsc.parallel_loop(0, 120, step=1, unroll=4)
        def fills(j, base=base, pre=pre, suf=suf):
            st[pl.ds(base + 16 * j, 16)] = pre
            st[pl.ds(base + 2176 + 16 * j, 16)] = suf

        @plsc.parallel_loop(0, 17, step=1, unroll=4)
        def body(j, base=base, tb=tb):
            st[pl.ds(base + 1919 + 16 * j, 16)] = tbl_v[pl.ds(tb + 16 * j, 16)]

        st[pl.ds(base + 2176, 16)] = suf

    def out_body(q2, carry):
        @pl.when(q2 >= 2)
        def _():
            pltpu.make_async_copy(out_hbm.at[0, pl.ds(0, 2)], stg.at[0], sem).wait()

        n = n0 + 2 * q2
        h = n // _NI
        I = n - _NI * h
        hh = h - h0
        W0 = hh * 4096 + 2047 - 8 * I
        b = lax.rem(q2, 2)

        @plsc.parallel_loop(0, 32, step=1, unroll=2)
        def sj(t):
            ii = t // 16
            J = t - 16 * ii
            w0 = W0 - 8 * ii + 128 * J
            for r in range(8):
                for u in range(8):
                    stg[b, ii, J, r, pl.ds(16 * u, 16)] = st[pl.ds(w0 - r + 16 * u, 16)]

        pltpu.async_copy(stg.at[b], out_hbm.at[h, pl.ds(I, 2)], sem)
        return carry

    lax.fori_loop(0, _PPW // 2, out_body, 0)
    pltpu.make_async_copy(out_hbm.at[0, pl.ds(0, 2)], stg.at[0], sem).wait()
    pltpu.make_async_copy(out_hbm.at[0, pl.ds(0, 2)], stg.at[0], sem).wait()


def kernel(seq_len, table):
    del seq_len
    out = _bias_kernel(table.T.reshape(_TTF))
    return out.transpose(1, 3, 2, 4, 0).reshape(_S, _S, _H)
